# baseline (speedup 1.0000x reference)
import functools

import jax
import jax.numpy as jnp
from jax import lax
from jax.experimental import pallas as pl
from jax.experimental.pallas import tpu as pltpu
from jax.experimental.pallas import tpu_sc as plsc

N = 10000
E = 320000
D = 128
B = 64
R = 16
L = 2
OUT = 64

NC = 2
NS = 16
NW = NC * NS
LANES = 16

NP = 10240
GCH = 80
GCHUNKS = NP // NW // GCH
SROWS = NP // NS

ECH = 64
EPW = 10240
EPAD = EPW * NW
NCHUNKS = EPW // ECH
TOTCH = EPAD // ECH
TROWS = 24

@functools.cache
def _sc_kernels():
    mesh = plsc.VectorSubcoreMesh(
        core_axis_name="c", subcore_axis_name="s", num_cores=NC,
        num_subcores=NS)

    @functools.partial(
        pl.kernel,
        mesh=mesh,
        out_type=jax.ShapeDtypeStruct((NP, D), jnp.float32),
        scratch_types=[
            pltpu.VMEM((GCH,), jnp.int32),
            pltpu.VMEM((GCH, D), jnp.float32),
            pltpu.SemaphoreType.DMA,
        ],
    )
    def _sc_gather(table_hbm, idx_hbm, out_hbm, idx_v, rows_v, sem):
        wid = lax.axis_index("s") * NC + lax.axis_index("c")
        base = wid * (GCH * GCHUNKS)

        def body(k, _):
            off = base + k * GCH
            pltpu.sync_copy(idx_hbm.at[pl.ds(off, GCH)], idx_v)
            pltpu.async_copy(table_hbm.at[idx_v], rows_v, sem).wait()
            pltpu.sync_copy(rows_v, out_hbm.at[pl.ds(off, GCH)])
            return 0

        lax.fori_loop(0, GCHUNKS, body, 0)

    @functools.partial(
        pl.kernel,
        mesh=mesh,
        out_type=jax.ShapeDtypeStruct((NC, NP, D), jnp.float32),
        scratch_types=[
            pltpu.VMEM((2, 2, ECH), jnp.int32),
            pltpu.VMEM((2, ECH), jnp.float32),
            pltpu.VMEM((2, ECH), jnp.int32),
            pltpu.VMEM((2, ECH, D), jnp.float32),
            pltpu.VMEM((2, ECH, D), jnp.float32),
            pltpu.VMEM((TROWS, D), jnp.float32),
            pltpu.VMEM_SHARED((NP, D), jnp.float32),
            pltpu.SemaphoreType.DMA,
            pltpu.SemaphoreType.DMA,
            pltpu.SemaphoreType.DMA,
            pltpu.SemaphoreType.DMA,
            pltpu.SemaphoreType.DMA,
            pltpu.SemaphoreType.DMA,
            pltpu.SemaphoreType.DMA,
            pltpu.SemaphoreType.DMA,
            pltpu.SemaphoreType.DMA,
            pltpu.SemaphoreType.DMA,
        ],
    )
    def _sc_layer(x_hbm, pk_hbm, at_hbm, ds_hbm, t_hbm, out_hbm,
                  pk_v, at_v, dst_r, xj_v, msg_v, t_v, agg_sh,
                  pk_s0, pk_s1, g_s0, g_s1, sc_s0, sc_s1, d_s0, d_s1,
                  g2_s0, g2_s1):
        cid = lax.axis_index("c")
        sid = lax.axis_index("s")
        wid = sid * NC + cid
        pk_sems = (pk_s0, pk_s1)
        g_sems = (g_s0, g_s1)
        g2_sems = (g2_s0, g2_s1)
        EH = ECH // 2
        sc_sems = (sc_s0, sc_s1)
        d_sems = (d_s0, d_s1)

        pltpu.sync_copy(t_hbm, t_v)

        def zrow(i, _):
            for j in range(D // LANES):
                msg_v[0, i, pl.ds(j * LANES, LANES)] = jnp.zeros(
                    (LANES,), jnp.float32)
            return 0

        lax.fori_loop(0, ECH, zrow, 0)
        stripe = sid * SROWS
        for k in range(SROWS // ECH):
            pltpu.sync_copy(msg_v.at[0],
                            agg_sh.at[pl.ds(stripe + k * ECH, ECH)])
        plsc.subcore_barrier()

        cbase = wid * NCHUNKS

        def start_pk(k, b):
            pltpu.async_copy(pk_hbm.at[cbase + k], pk_v.at[b], pk_sems[b])
            pltpu.async_copy(at_hbm.at[cbase + k], at_v.at[b], pk_sems[b])

        def wait_pk(b):
            pltpu.make_async_copy(pk_hbm.at[0], pk_v.at[b],
                                  pk_sems[b]).wait()
            pltpu.make_async_copy(at_hbm.at[0], at_v.at[b],
                                  pk_sems[b]).wait()

        def start_g(b):
            pltpu.async_copy(x_hbm.at[pk_v.at[b, 0, pl.ds(0, EH)]],
                             xj_v.at[b, pl.ds(0, EH)], g_sems[b])
            pltpu.async_copy(x_hbm.at[pk_v.at[b, 0, pl.ds(EH, EH)]],
                             xj_v.at[b, pl.ds(EH, EH)], g2_sems[b])

        def wait_g(b):
            pltpu.make_async_copy(x_hbm.at[pk_v.at[b, 0, pl.ds(0, EH)]],
                                  xj_v.at[b, pl.ds(0, EH)], g_sems[b]).wait()
            pltpu.make_async_copy(x_hbm.at[pk_v.at[b, 0, pl.ds(EH, EH)]],
                                  xj_v.at[b, pl.ds(EH, EH)],
                                  g2_sems[b]).wait()

        def start_d(k, b):
            pltpu.async_copy(ds_hbm.at[cbase + k], dst_r.at[b], d_sems[b])

        def wait_d(b):
            pltpu.make_async_copy(ds_hbm.at[0], dst_r.at[b],
                                  d_sems[b]).wait()

        def start_sc(b):
            pltpu.async_copy(msg_v.at[b], agg_sh.at[dst_r.at[b]], sc_sems[b],
                             add=True)

        def wait_sc(b):
            pltpu.make_async_copy(msg_v.at[b], agg_sh.at[dst_r.at[b]],
                                  sc_sems[b]).wait()

        UR = 8

        def compute(b):
            def grp(g, _):
                base16 = g * LANES
                avec = at_v[b, pl.ds(base16, LANES)]
                rvec = pk_v[b, 1, pl.ds(base16, LANES)]
                NJ = D // LANES

                for u0 in range(0, LANES, UR):
                    als = [avec[u0 + t] for t in range(UR)]
                    rls = [rvec[u0 + t] for t in range(UR)]
                    rows = [base16 + u0 + t for t in range(UR)]

                    def loads(j):
                        sl = pl.ds(j * LANES, LANES)
                        xs = [xj_v[b, rows[t], sl] for t in range(UR)]
                        ts = [t_v[rls[t], sl] for t in range(UR)]
                        return xs, ts

                    stage = [loads(0), loads(1)]
                    for j in range(NJ):
                        if j + 2 < NJ:
                            stage.append(loads(j + 2))
                        xs, ts = stage[j]
                        sl = pl.ds(j * LANES, LANES)
                        res = [jnp.maximum(xs[t] * als[t] + ts[t], 0.0)
                               for t in range(UR)]
                        for t in range(UR):
                            msg_v[b, rows[t], sl] = res[t]
                return 0

            lax.fori_loop(0, ECH // LANES, grp, 0)

        def step(k, b, first=False, prefetch_g=True, prefetch_pk=True):
            nb = 1 - b
            wait_g(b)
            if not first:
                wait_sc(nb)
                start_d(k + 1, nb)
            if prefetch_g:
                wait_pk(nb)
                start_g(nb)
            compute(b)
            wait_d(b)
            start_sc(b)
            if prefetch_pk:
                start_pk(k + 2, b)

        start_pk(0, 0)
        start_pk(1, 1)
        start_d(0, 0)
        start_d(1, 1)
        wait_pk(0)
        start_g(0)

        step(0, 0, first=True)
        step(1, 1)

        def pair(g, _):
            step(2 * g, 0)
            step(2 * g + 1, 1)
            return 0

        lax.fori_loop(1, (NCHUNKS - 2) // 2, pair, 0)
        wait_g(0)
        wait_sc(1)
        start_d(NCHUNKS - 1, 1)
        wait_pk(1)
        start_g(1)
        compute(0)
        wait_d(0)
        start_sc(0)
        wait_g(1)
        wait_sc(0)
        compute(1)
        wait_d(1)
        start_sc(1)
        wait_sc(1)
        plsc.subcore_barrier()

        for k in range(SROWS // ECH):
            pltpu.sync_copy(agg_sh.at[pl.ds(stripe + k * ECH, ECH)],
                            msg_v.at[0])
            pltpu.sync_copy(msg_v.at[0],
                            out_hbm.at[cid, pl.ds(stripe + k * ECH, ECH)])

    return _sc_gather, _sc_layer


def _prep_body(rel_emb_ref, wr_ref, br_ref, t_ref):
    re = rel_emb_ref[...]
    for l in range(L):
        w = jnp.dot(re, wr_ref[l], preferred_element_type=jnp.float32)
        w = w + br_ref[l, 0]
        t = w * re
        t_ref[l] = jnp.concatenate(
            [t, jnp.zeros((TROWS - R, D), jnp.float32)], axis=0)


def _tc_prep(rel_emb, WR, bR):
    return pl.pallas_call(
        _prep_body,
        out_shape=jax.ShapeDtypeStruct((L, TROWS, D), jnp.float32),
    )(rel_emb, WR, bR)


def _layer_body(part_ref, x_ref, w1_ref, b1_ref, w2_ref, b2_ref, eps_ref,
                out_ref):
    agg = part_ref[0] + part_ref[1]
    x = x_ref[...]
    out = agg + (1.0 + eps_ref[0, 0]) * x
    h = jnp.maximum(
        jnp.dot(out, w1_ref[...], preferred_element_type=jnp.float32)
        + b1_ref[...], 0.0)
    out_ref[...] = (
        jnp.dot(h, w2_ref[...], preferred_element_type=jnp.float32)
        + b2_ref[...])


def _tc_layer(part, x, w1, b1, w2, b2, eps_l):
    return pl.pallas_call(
        _layer_body,
        out_shape=jax.ShapeDtypeStruct((NP, D), jnp.float32),
    )(part, x, w1, b1.reshape(1, D), w2, b2.reshape(1, D),
      eps_l.reshape(1, 1))


def _epi_body(x_ref, batch_ref, cb1_ref, cw2_ref, cb2_ref, gw1_ref, gb1_ref,
              gw2_ref, gb2_ref, wo_ref, bo_ref, out_ref):
    x = x_ref[...]
    c = jax.nn.sigmoid(
        jnp.dot(jnp.maximum(cb1_ref[...], 0.0), cw2_ref[...],
                preferred_element_type=jnp.float32)[0, 0] + cb2_ref[0, 0])
    geff = c * gw1_ref[:D] + gw1_ref[D:]
    gi = jnp.maximum(
        jnp.dot(x, geff, preferred_element_type=jnp.float32)
        + gb1_ref[...], 0.0)
    gate = jax.nn.sigmoid(
        jnp.dot(gi, gw2_ref[...], preferred_element_type=jnp.float32)
        + gb2_ref[0, 0])
    fused = x * (1.0 - gate * (1.0 - c))
    onehot = (batch_ref[...] ==
              lax.broadcasted_iota(jnp.int32, (NP, B), 1)).astype(jnp.float32)
    sums = lax.dot_general(onehot, fused, (((0,), (0,)), ((), ())),
                           preferred_element_type=jnp.float32)
    counts = lax.dot_general(onehot, jnp.ones((NP, D), jnp.float32),
                             (((0,), (0,)), ((), ())),
                             preferred_element_type=jnp.float32)
    pooled = sums / jnp.maximum(counts, 1.0)
    out_ref[...] = (
        jnp.dot(pooled, wo_ref[...], preferred_element_type=jnp.float32)
        + bo_ref[...])


def _tc_epilogue(x, batch, Cb1, Cw2, Cb2, Gw1, Gb1, Gw2, Gb2, Wo, bo):
    return pl.pallas_call(
        _epi_body,
        out_shape=jax.ShapeDtypeStruct((B, OUT), jnp.float32),
    )(x, batch.reshape(NP, 1), Cb1.reshape(1, D), Cw2, Cb2.reshape(1, 1),
      Gw1, Gb1.reshape(1, D), Gw2, Gb2.reshape(1, 1), Wo,
      bo.reshape(1, OUT))


def kernel(node_ids, edge_index, rel_ids, batch, attn, node_emb, rel_emb,
           W1, b1, W2, b2, WR, bR, eps,
           Cw1, Cb1, Cw2, Cb2, Gw1, Gb1, Gw2, Gb2, Wo, bo):
    ids_pad = jnp.concatenate(
        [node_ids, jnp.zeros((NP - N,), jnp.int32)])
    batch_pad = jnp.concatenate(
        [batch, jnp.full((NP - N,), B, jnp.int32)])
    src = jnp.concatenate(
        [edge_index[0], jnp.zeros((EPAD - E,), jnp.int32)])
    dst = jnp.concatenate(
        [edge_index[1], jnp.zeros((EPAD - E,), jnp.int32)])
    rel = jnp.concatenate(
        [rel_ids, jnp.full((EPAD - E,), R, jnp.int32)])
    attn_f = jnp.concatenate(
        [attn[:, 0], jnp.zeros((EPAD - E,), jnp.float32)]
        ).reshape(TOTCH, ECH)
    pk = jnp.stack([src.reshape(TOTCH, ECH), rel.reshape(TOTCH, ECH)],
                   axis=1)
    dst_c = dst.reshape(TOTCH, ECH)

    t_tab = _tc_prep(rel_emb, WR, bR)

    sc_gather, sc_layer = _sc_kernels()
    x = sc_gather(node_emb, ids_pad)
    for l in range(L):
        part = sc_layer(x, pk, attn_f, dst_c, t_tab[l])
        x = _tc_layer(part, x, W1[l], b1[l], W2[l], b2[l], eps[l])

    return _tc_epilogue(x, batch_pad, Cb1, Cw2, Cb2, Gw1, Gb1, Gw2, Gb2, Wo, bo)


# speedup vs baseline: 1.0009x; 1.0009x over previous
import functools

import jax
import jax.numpy as jnp
from jax import lax
from jax.experimental import pallas as pl
from jax.experimental.pallas import tpu as pltpu
from jax.experimental.pallas import tpu_sc as plsc

N = 10000
E = 320000
D = 128
B = 64
R = 16
L = 2
OUT = 64

NC = 2
NS = 16
NW = NC * NS
LANES = 16

NP = 10240
GCH = 80
GCHUNKS = NP // NW // GCH
SROWS = NP // NS

ECH = 64
EPW = 10240
EPAD = EPW * NW
NCHUNKS = EPW // ECH
TOTCH = EPAD // ECH
TROWS = 24

@functools.cache
def _sc_kernels():
    mesh = plsc.VectorSubcoreMesh(
        core_axis_name="c", subcore_axis_name="s", num_cores=NC,
        num_subcores=NS)

    @functools.partial(
        pl.kernel,
        mesh=mesh,
        out_type=jax.ShapeDtypeStruct((NP, D), jnp.float32),
        scratch_types=[
            pltpu.VMEM((GCH,), jnp.int32),
            pltpu.VMEM((GCH, D), jnp.float32),
            pltpu.SemaphoreType.DMA,
        ],
    )
    def _sc_gather(table_hbm, idx_hbm, out_hbm, idx_v, rows_v, sem):
        wid = lax.axis_index("s") * NC + lax.axis_index("c")
        base = wid * (GCH * GCHUNKS)

        def body(k, _):
            off = base + k * GCH
            pltpu.sync_copy(idx_hbm.at[pl.ds(off, GCH)], idx_v)
            pltpu.async_copy(table_hbm.at[idx_v], rows_v, sem).wait()
            pltpu.sync_copy(rows_v, out_hbm.at[pl.ds(off, GCH)])
            return 0

        lax.fori_loop(0, GCHUNKS, body, 0)

    @functools.partial(
        pl.kernel,
        mesh=mesh,
        out_type=jax.ShapeDtypeStruct((NC, NP, D), jnp.float32),
        scratch_types=[
            pltpu.VMEM((2, 2, ECH), jnp.int32),
            pltpu.VMEM((2, ECH), jnp.float32),
            pltpu.VMEM((2, ECH), jnp.int32),
            pltpu.VMEM((2, ECH, D), jnp.float32),
            pltpu.VMEM((2, ECH, D), jnp.float32),
            pltpu.VMEM((TROWS, D), jnp.float32),
            pltpu.VMEM_SHARED((NP, D), jnp.float32),
            pltpu.SemaphoreType.DMA,
            pltpu.SemaphoreType.DMA,
            pltpu.SemaphoreType.DMA,
            pltpu.SemaphoreType.DMA,
            pltpu.SemaphoreType.DMA,
            pltpu.SemaphoreType.DMA,
            pltpu.SemaphoreType.DMA,
            pltpu.SemaphoreType.DMA,
        ],
    )
    def _sc_layer(x_hbm, pk_hbm, at_hbm, ds_hbm, t_hbm, out_hbm,
                  pk_v, at_v, dst_r, xj_v, msg_v, t_v, agg_sh,
                  pk_s0, pk_s1, g_s0, g_s1, sc_s0, sc_s1, d_s0, d_s1):
        cid = lax.axis_index("c")
        sid = lax.axis_index("s")
        wid = sid * NC + cid
        pk_sems = (pk_s0, pk_s1)
        g_sems = (g_s0, g_s1)
        sc_sems = (sc_s0, sc_s1)
        d_sems = (d_s0, d_s1)

        pltpu.sync_copy(t_hbm, t_v)

        def zrow(i, _):
            for j in range(D // LANES):
                msg_v[0, i, pl.ds(j * LANES, LANES)] = jnp.zeros(
                    (LANES,), jnp.float32)
            return 0

        lax.fori_loop(0, ECH, zrow, 0)
        stripe = sid * SROWS
        for k in range(SROWS // ECH):
            pltpu.sync_copy(msg_v.at[0],
                            agg_sh.at[pl.ds(stripe + k * ECH, ECH)])
        plsc.subcore_barrier()

        cbase = wid * NCHUNKS

        def start_pk(k, b):
            pltpu.async_copy(pk_hbm.at[cbase + k], pk_v.at[b], pk_sems[b])
            pltpu.async_copy(at_hbm.at[cbase + k], at_v.at[b], pk_sems[b])

        def wait_pk(b):
            pltpu.make_async_copy(pk_hbm.at[0], pk_v.at[b],
                                  pk_sems[b]).wait()
            pltpu.make_async_copy(at_hbm.at[0], at_v.at[b],
                                  pk_sems[b]).wait()

        def start_g(b):
            pltpu.async_copy(x_hbm.at[pk_v.at[b, 0]], xj_v.at[b], g_sems[b])

        def wait_g(b):
            pltpu.make_async_copy(x_hbm.at[pk_v.at[b, 0]], xj_v.at[b],
                                  g_sems[b]).wait()

        def start_d(k, b):
            pltpu.async_copy(ds_hbm.at[cbase + k], dst_r.at[b], d_sems[b])

        def wait_d(b):
            pltpu.make_async_copy(ds_hbm.at[0], dst_r.at[b],
                                  d_sems[b]).wait()

        def start_sc(b):
            pltpu.async_copy(msg_v.at[b], agg_sh.at[dst_r.at[b]], sc_sems[b],
                             add=True)

        def wait_sc(b):
            pltpu.make_async_copy(msg_v.at[b], agg_sh.at[dst_r.at[b]],
                                  sc_sems[b]).wait()

        UR = 8

        def compute(b):
            def grp(g, _):
                base16 = g * LANES
                avec = at_v[b, pl.ds(base16, LANES)]
                rvec = pk_v[b, 1, pl.ds(base16, LANES)]
                NJ = D // LANES

                for u0 in range(0, LANES, UR):
                    als = [avec[u0 + t] for t in range(UR)]
                    rls = [rvec[u0 + t] for t in range(UR)]
                    rows = [base16 + u0 + t for t in range(UR)]

                    def loads(j):
                        sl = pl.ds(j * LANES, LANES)
                        xs = [xj_v[b, rows[t], sl] for t in range(UR)]
                        ts = [t_v[rls[t], sl] for t in range(UR)]
                        return xs, ts

                    stage = [loads(0), loads(1)]
                    for j in range(NJ):
                        if j + 2 < NJ:
                            stage.append(loads(j + 2))
                        xs, ts = stage[j]
                        sl = pl.ds(j * LANES, LANES)
                        res = [jnp.maximum(xs[t] * als[t] + ts[t], 0.0)
                               for t in range(UR)]
                        for t in range(UR):
                            msg_v[b, rows[t], sl] = res[t]
                return 0

            lax.fori_loop(0, ECH // LANES, grp, 0)

        def step(k, b, first=False, prefetch_g=True, prefetch_pk=True):
            nb = 1 - b
            wait_g(b)
            if not first:
                wait_sc(nb)
                start_d(k + 1, nb)
            if prefetch_g:
                wait_pk(nb)
                start_g(nb)
            compute(b)
            wait_d(b)
            start_sc(b)
            if prefetch_pk:
                start_pk(k + 2, b)

        start_pk(0, 0)
        start_pk(1, 1)
        start_d(0, 0)
        start_d(1, 1)
        wait_pk(0)
        start_g(0)

        step(0, 0, first=True)
        step(1, 1)

        def pair(g, _):
            step(2 * g, 0)
            step(2 * g + 1, 1)
            return 0

        lax.fori_loop(1, (NCHUNKS - 2) // 2, pair, 0)
        wait_g(0)
        wait_sc(1)
        start_d(NCHUNKS - 1, 1)
        wait_pk(1)
        start_g(1)
        compute(0)
        wait_d(0)
        start_sc(0)
        wait_g(1)
        wait_sc(0)
        compute(1)
        wait_d(1)
        start_sc(1)
        wait_sc(1)
        plsc.subcore_barrier()

        for k in range(SROWS // ECH):
            pltpu.sync_copy(agg_sh.at[pl.ds(stripe + k * ECH, ECH)],
                            msg_v.at[0])
            pltpu.sync_copy(msg_v.at[0],
                            out_hbm.at[cid, pl.ds(stripe + k * ECH, ECH)])

    return _sc_gather, _sc_layer


def _prep_body(rel_emb_ref, wr_ref, br_ref, t_ref):
    re = rel_emb_ref[...]
    for l in range(L):
        w = jnp.dot(re, wr_ref[l], preferred_element_type=jnp.float32)
        w = w + br_ref[l, 0]
        t = w * re
        t_ref[l] = jnp.concatenate(
            [t, jnp.zeros((TROWS - R, D), jnp.float32)], axis=0)


def _tc_prep(rel_emb, WR, bR):
    return pl.pallas_call(
        _prep_body,
        out_shape=jax.ShapeDtypeStruct((L, TROWS, D), jnp.float32),
    )(rel_emb, WR, bR)


def _layer_body(part_ref, x_ref, w1_ref, b1_ref, w2_ref, b2_ref, eps_ref,
                out_ref):
    agg = part_ref[0] + part_ref[1]
    x = x_ref[...]
    out = agg + (1.0 + eps_ref[0, 0]) * x
    h = jnp.maximum(
        jnp.dot(out, w1_ref[...], preferred_element_type=jnp.float32)
        + b1_ref[...], 0.0)
    out_ref[...] = (
        jnp.dot(h, w2_ref[...], preferred_element_type=jnp.float32)
        + b2_ref[...])


def _tc_layer(part, x, w1, b1, w2, b2, eps_l):
    return pl.pallas_call(
        _layer_body,
        out_shape=jax.ShapeDtypeStruct((NP, D), jnp.float32),
    )(part, x, w1, b1.reshape(1, D), w2, b2.reshape(1, D),
      eps_l.reshape(1, 1))


def _epi_body(x_ref, batch_ref, cb1_ref, cw2_ref, cb2_ref, gw1_ref, gb1_ref,
              gw2_ref, gb2_ref, wo_ref, bo_ref, out_ref):
    x = x_ref[...]
    c = jax.nn.sigmoid(
        jnp.dot(jnp.maximum(cb1_ref[...], 0.0), cw2_ref[...],
                preferred_element_type=jnp.float32)[0, 0] + cb2_ref[0, 0])
    geff = c * gw1_ref[:D] + gw1_ref[D:]
    gi = jnp.maximum(
        jnp.dot(x, geff, preferred_element_type=jnp.float32)
        + gb1_ref[...], 0.0)
    gate = jax.nn.sigmoid(
        jnp.dot(gi, gw2_ref[...], preferred_element_type=jnp.float32)
        + gb2_ref[0, 0])
    fused = x * (1.0 - gate * (1.0 - c))
    onehot = (batch_ref[...] ==
              lax.broadcasted_iota(jnp.int32, (NP, B), 1)).astype(jnp.float32)
    sums = lax.dot_general(onehot, fused, (((0,), (0,)), ((), ())),
                           preferred_element_type=jnp.float32)
    counts = lax.dot_general(onehot, jnp.ones((NP, D), jnp.float32),
                             (((0,), (0,)), ((), ())),
                             preferred_element_type=jnp.float32)
    pooled = sums / jnp.maximum(counts, 1.0)
    out_ref[...] = (
        jnp.dot(pooled, wo_ref[...], preferred_element_type=jnp.float32)
        + bo_ref[...])


def _tc_epilogue(x, batch, Cb1, Cw2, Cb2, Gw1, Gb1, Gw2, Gb2, Wo, bo):
    return pl.pallas_call(
        _epi_body,
        out_shape=jax.ShapeDtypeStruct((B, OUT), jnp.float32),
    )(x, batch.reshape(NP, 1), Cb1.reshape(1, D), Cw2, Cb2.reshape(1, 1),
      Gw1, Gb1.reshape(1, D), Gw2, Gb2.reshape(1, 1), Wo,
      bo.reshape(1, OUT))


def kernel(node_ids, edge_index, rel_ids, batch, attn, node_emb, rel_emb,
           W1, b1, W2, b2, WR, bR, eps,
           Cw1, Cb1, Cw2, Cb2, Gw1, Gb1, Gw2, Gb2, Wo, bo):
    ids_pad = jnp.concatenate(
        [node_ids, jnp.zeros((NP - N,), jnp.int32)])
    batch_pad = jnp.concatenate(
        [batch, jnp.full((NP - N,), B, jnp.int32)])
    src = jnp.concatenate(
        [edge_index[0], jnp.zeros((EPAD - E,), jnp.int32)])
    dst = jnp.concatenate(
        [edge_index[1], jnp.zeros((EPAD - E,), jnp.int32)])
    rel = jnp.concatenate(
        [rel_ids, jnp.full((EPAD - E,), R, jnp.int32)])
    attn_f = jnp.concatenate(
        [attn[:, 0], jnp.zeros((EPAD - E,), jnp.float32)]
        ).reshape(TOTCH, ECH)
    pk = jnp.stack([src.reshape(TOTCH, ECH), rel.reshape(TOTCH, ECH)],
                   axis=1)
    dst_c = dst.reshape(TOTCH, ECH)

    t_tab = _tc_prep(rel_emb, WR, bR)

    sc_gather, sc_layer = _sc_kernels()
    x = sc_gather(node_emb, ids_pad)
    for l in range(L):
        part = sc_layer(x, pk, attn_f, dst_c, t_tab[l])
        x = _tc_layer(part, x, W1[l], b1[l], W2[l], b2[l], eps[l])

    return _tc_epilogue(x, batch_pad, Cb1, Cw2, Cb2, Gw1, Gb1, Gw2, Gb2, Wo, bo)
